# baseline (speedup 1.0000x reference)
import jax
import jax.numpy as jnp
from jax import lax
from jax.experimental import pallas as pl
from jax.experimental.pallas import tpu as pltpu
from jax.experimental.pallas import tpu_sc as plsc

_N_ROWS = 4194304
_C = 5
_NC = 2
_NS = 16
_NW = _NC * _NS
_ROWS_W = _N_ROWS // _NW
_CHUNK_ROWS = 8192
_NCHUNK = _ROWS_W // _CHUNK_ROWS
_CHUNK_FLAT = _CHUNK_ROWS * _C
_L = 16


def _sc_body(x_hbm, out_hbm, inbuf, outbuf):
    wid = lax.axis_index("s") * _NC + lax.axis_index("c")
    row0_w = wid * _ROWS_W
    lane_idx = lax.iota(jnp.int32, _L) * _C
    zero = jnp.zeros((_L,), jnp.float32)
    half = jnp.full((_L,), 0.5, jnp.float32)

    def chunk_body(ci, carry):
        row0 = row0_w + ci * _CHUNK_ROWS
        pltpu.sync_copy(x_hbm.at[pl.ds(row0 * _C, _CHUNK_FLAT)], inbuf)

        def grp(i, c2):
            i0 = lane_idx + i * (_L * _C)
            t = plsc.load_gather(inbuf, [i0])
            loc = plsc.load_gather(inbuf, [i0 + 1])
            bsc = plsc.load_gather(inbuf, [i0 + 2])
            tsc = plsc.load_gather(inbuf, [i0 + 3])
            tsh = plsc.load_gather(inbuf, [i0 + 4])
            ti = t.astype(jnp.int32)
            aff = loc * tsc + tsh
            ex = jnp.exp(loc + half * (bsc * bsc))
            res = jnp.where(ti == 0, aff, jnp.where(ti == 1, ex, zero))
            outbuf[pl.ds(i * _L, _L)] = res
            return c2

        lax.fori_loop(0, _CHUNK_ROWS // _L, grp, 0)
        pltpu.sync_copy(outbuf, out_hbm.at[pl.ds(row0, _CHUNK_ROWS)])
        return carry

    lax.fori_loop(0, _NCHUNK, chunk_body, 0)


def kernel(x):
    flat = x.reshape(-1)
    f = pl.kernel(
        _sc_body,
        out_type=jax.ShapeDtypeStruct((_N_ROWS,), jnp.float32),
        mesh=plsc.VectorSubcoreMesh(core_axis_name="c", subcore_axis_name="s"),
        scratch_types=[
            pltpu.VMEM((_CHUNK_FLAT,), jnp.float32),
            pltpu.VMEM((_CHUNK_ROWS,), jnp.float32),
        ],
        compiler_params=pltpu.CompilerParams(needs_layout_passes=False),
    )
    return f(flat)


# speedup vs baseline: 1.0780x; 1.0780x over previous
import jax
import jax.numpy as jnp
from jax import lax
from jax.experimental import pallas as pl
from jax.experimental.pallas import tpu as pltpu
from jax.experimental.pallas import tpu_sc as plsc

_N_ROWS = 4194304
_C = 5
_NC = 2
_NS = 16
_NW = _NC * _NS
_ROWS_W = _N_ROWS // _NW
_CHUNK_ROWS = 8192
_NCHUNK = _ROWS_W // _CHUNK_ROWS
_CHUNK_FLAT = _CHUNK_ROWS * _C
_L = 16


def _sc_body(x_hbm, out_hbm, in0, in1, ot0, ot1,
             isem0, isem1, osem0, osem1):
    wid = lax.axis_index("s") * _NC + lax.axis_index("c")
    row0_w = wid * _ROWS_W
    lane_idx = lax.iota(jnp.int32, _L) * _C
    zero = jnp.zeros((_L,), jnp.float32)
    half = jnp.full((_L,), 0.5, jnp.float32)

    inbufs = (in0, in1)
    outbufs = (ot0, ot1)
    isems = (isem0, isem1)
    osems = (osem0, osem1)

    def in_copy(ci):
        row0 = row0_w + ci * _CHUNK_ROWS
        b = ci % 2
        return pltpu.make_async_copy(
            x_hbm.at[pl.ds(row0 * _C, _CHUNK_FLAT)], inbufs[b], isems[b])

    def out_copy(ci):
        row0 = row0_w + ci * _CHUNK_ROWS
        b = ci % 2
        return pltpu.make_async_copy(
            outbufs[b], out_hbm.at[pl.ds(row0, _CHUNK_ROWS)], osems[b])

    in_copy(0).start()
    for ci in range(_NCHUNK):
        b = ci % 2
        if ci + 1 < _NCHUNK:
            in_copy(ci + 1).start()
        in_copy(ci).wait()
        if ci >= 2:
            out_copy(ci - 2).wait()
        inb = inbufs[b]
        otb = outbufs[b]

        @plsc.parallel_loop(0, _CHUNK_ROWS // _L, unroll=8)
        def _grp(i):
            i0 = lane_idx + i * (_L * _C)
            t = plsc.load_gather(inb, [i0])
            loc = plsc.load_gather(inb, [i0 + 1])
            bsc = plsc.load_gather(inb, [i0 + 2])
            tsc = plsc.load_gather(inb, [i0 + 3])
            tsh = plsc.load_gather(inb, [i0 + 4])
            ti = t.astype(jnp.int32)
            aff = loc * tsc + tsh
            ex = jnp.exp(loc + half * (bsc * bsc))
            res = jnp.where(ti == 0, aff, jnp.where(ti == 1, ex, zero))
            otb[pl.ds(i * _L, _L)] = res

        out_copy(ci).start()
    out_copy(_NCHUNK - 2).wait()
    out_copy(_NCHUNK - 1).wait()


def kernel(x):
    flat = x.reshape(-1)
    f = pl.kernel(
        _sc_body,
        out_type=jax.ShapeDtypeStruct((_N_ROWS,), jnp.float32),
        mesh=plsc.VectorSubcoreMesh(core_axis_name="c", subcore_axis_name="s"),
        scratch_types=[
            pltpu.VMEM((_CHUNK_FLAT,), jnp.float32),
            pltpu.VMEM((_CHUNK_FLAT,), jnp.float32),
            pltpu.VMEM((_CHUNK_ROWS,), jnp.float32),
            pltpu.VMEM((_CHUNK_ROWS,), jnp.float32),
            pltpu.SemaphoreType.DMA,
            pltpu.SemaphoreType.DMA,
            pltpu.SemaphoreType.DMA,
            pltpu.SemaphoreType.DMA,
        ],
        compiler_params=pltpu.CompilerParams(needs_layout_passes=False),
    )
    return f(flat)


# speedup vs baseline: 1.0782x; 1.0002x over previous
import jax
import jax.numpy as jnp
from jax import lax
from jax.experimental import pallas as pl
from jax.experimental.pallas import tpu as pltpu
from jax.experimental.pallas import tpu_sc as plsc

_N_ROWS = 4194304
_C = 5
_NC = 2
_NS = 16
_NW = _NC * _NS
_FLAT = _N_ROWS * _C
_W = 128
_HROWS = _FLAT // _W
_HROWS_W = _HROWS // _NW
_CHUNK_H = 80
_NCHUNK = _HROWS_W // _CHUNK_H
_CHUNK_FLAT = _CHUNK_H * _W
_OW = 128
_OROWS = _N_ROWS // _OW
_OROWS_W = _OROWS // _NW
_OCHUNK = _CHUNK_FLAT // _C // _OW
_L = 16
_NBUF = 4
_NGRP = _CHUNK_FLAT // _C // _L


def _sc_body(x_hbm, out_hbm, inb, otb, idxi, idxo, isem, osem):
    cid = lax.axis_index("c")
    sid = lax.axis_index("s")
    wid = sid * _NC + cid
    iota = lax.iota(jnp.int32, _L)
    lane5 = iota * _C
    zero = jnp.zeros((_L,), jnp.float32)
    half = jnp.full((_L,), 0.5, jnp.float32)
    irow0 = wid * _HROWS_W
    orow0 = wid * _OROWS_W

    def start_in(ci, b):
        base = irow0 + ci * _CHUNK_H
        ib = idxi.at[b]
        for k in range(_CHUNK_H // _L):
            ib[pl.ds(k * _L, _L)] = base + k * _L + iota
        return pltpu.make_async_copy(x_hbm.at[ib], inb.at[b], isem.at[b]).start()

    def start_out(ci, b):
        base = orow0 + ci * _OCHUNK
        ob = idxo.at[b]
        for k in range(_OCHUNK // _L):
            ob[pl.ds(k * _L, _L)] = base + k * _L + iota
        return pltpu.make_async_copy(otb.at[b], out_hbm.at[ob], osem.at[b]).start()

    def wait_in(b):
        pltpu.make_async_copy(x_hbm.at[idxi.at[b]], inb.at[b], isem.at[b]).wait()

    def wait_out(b):
        pltpu.make_async_copy(otb.at[b], out_hbm.at[idxo.at[b]], osem.at[b]).wait()

    def compute(b):
        src = inb.at[b]
        dst = otb.at[b]

        @plsc.parallel_loop(0, _NGRP, unroll=8)
        def _grp(g):
            w0 = lane5 + g * (_L * _C)
            vals = []
            for c in range(_C):
                w = w0 + c
                vals.append(plsc.load_gather(src, [w >> 7, w & 127]))
            t, loc, bsc, tsc, tsh = vals
            ti = t.astype(jnp.int32)
            aff = loc * tsc + tsh
            ex = jnp.exp(loc + half * (bsc * bsc))
            res = jnp.where(ti == 0, aff, jnp.where(ti == 1, ex, zero))
            dst[g >> 3, pl.ds((g & 7) * _L, _L)] = res

    for ci in range(_NBUF - 1):
        start_in(ci, ci)

    def chunk_iter(it, carry):
        for b in range(_NBUF):
            ci = it * _NBUF + b

            @pl.when(ci + _NBUF - 1 < _NCHUNK)
            def _():
                start_in(ci + _NBUF - 1, (b - 1) % _NBUF)

            wait_in(b)

            @pl.when(ci >= _NBUF)
            def _():
                wait_out(b)

            compute(b)
            start_out(ci, b)
        return carry

    lax.fori_loop(0, _NCHUNK // _NBUF, chunk_iter, 0)
    for b in range(_NBUF):
        wait_out(b)


def kernel(x):
    x2 = x.reshape(_HROWS, _W)
    f = pl.kernel(
        _sc_body,
        out_type=jax.ShapeDtypeStruct((_OROWS, _OW), jnp.float32),
        mesh=plsc.VectorSubcoreMesh(core_axis_name="c", subcore_axis_name="s"),
        scratch_types=[
            pltpu.VMEM((_NBUF, _CHUNK_H, _W), jnp.float32),
            pltpu.VMEM((_NBUF, _OCHUNK, _OW), jnp.float32),
            pltpu.VMEM((_NBUF, _CHUNK_H), jnp.int32),
            pltpu.VMEM((_NBUF, _OCHUNK), jnp.int32),
            pltpu.SemaphoreType.DMA((_NBUF,)),
            pltpu.SemaphoreType.DMA((_NBUF,)),
        ],
        compiler_params=pltpu.CompilerParams(needs_layout_passes=False),
    )
    return f(x2).reshape(_N_ROWS)
